# baseline (speedup 1.0000x reference)
import jax
import jax.numpy as jnp
from jax.experimental import pallas as pl
from jax.experimental.pallas import tpu as pltpu

_SMAX = 32
_NSTATIC = 3


def _pick_block(n):
    for cand in (5120, 2560, 1280, 640, 320, 160, 80, 40, 16, 8):
        if n % cand == 0:
            return cand
    return n


def _body(ptr_ref, ptrw_ref, x_ref, h_ref, w_ref, b_ref, out_ref, *, K, F, D, B):
    k = pl.program_id(0)
    r0 = k * K
    z = jnp.dot(
        x_ref[...].astype(jnp.bfloat16),
        w_ref[:F].astype(jnp.bfloat16),
        preferred_element_type=jnp.float32,
    )
    z = z + jax.lax.dot_general(
        h_ref[...], w_ref[F:],
        dimension_numbers=(((0,), (0,)), ((), ())),
        preferred_element_type=jnp.float32,
    )
    z = z + b_ref[...]
    z = jnp.where(z >= 0, z, 0.2 * z)

    ptr_wide = ptrw_ref[...]

    def seg_of(r):
        return jnp.sum((ptr_wide <= r).astype(jnp.int32)) - 1

    s0 = seg_of(r0)
    s1 = seg_of(r0 + K - 1)
    nchunks = (s1 - s0 + _SMAX) // _SMAX

    zstart = seg_of(r0 - 1) + 1
    zend = s0 + jnp.maximum(nchunks, _NSTATIC) * _SMAX
    nz = (zend - zstart + _SMAX - 1) // _SMAX
    zzero = jnp.zeros((_SMAX, D), jnp.float32)

    def zchunk(m, carry):
        out_ref[pl.ds(zstart + m * _SMAX, _SMAX), :] = zzero
        return carry

    for m in range(_NSTATIC + 1):
        zchunk(m, 0)
    jax.lax.fori_loop(_NSTATIC + 1, nz, zchunk, 0)

    rows = r0 + jax.lax.broadcasted_iota(jnp.int32, (_SMAX + 1, K), 1)
    zb = z.astype(jnp.bfloat16)

    def chunk(j, carry):
        base = s0 + j * _SMAX
        bounds = ptr_ref[pl.ds(base, _SMAX + 1), :]
        step = (rows >= bounds).astype(jnp.bfloat16)
        csum = jnp.dot(step, zb, preferred_element_type=jnp.float32)
        part = csum[:_SMAX] - csum[1:]
        out_ref[pl.ds(base, _SMAX), :] += part
        return carry

    for j in range(_NSTATIC):
        chunk(j, 0)
    jax.lax.fori_loop(_NSTATIC, nchunks, chunk, 0)


def kernel(h_node, x, ptr, W, b):
    N, F = x.shape
    D = h_node.shape[1]
    B = ptr.shape[0] - 1
    K = _pick_block(N)
    G = N // K

    pad_rows = (_NSTATIC + 1) * _SMAX
    B_pad = -(B + pad_rows) % 8 + (B + pad_rows)
    P = -(B + 1 + pad_rows) % 8 + (B + 1 + pad_rows)
    ptr_pad = jnp.concatenate(
        [ptr.astype(jnp.int32), jnp.full((P - (B + 1),), N, jnp.int32)]
    ).reshape(P, 1)
    Pw = -(B + 1) % 1024 + (B + 1)
    ptr_wide = jnp.concatenate(
        [ptr.astype(jnp.int32), jnp.full((Pw - (B + 1),), N, jnp.int32)]
    ).reshape(Pw // 128, 128)

    out = pl.pallas_call(
        lambda *refs: _body(*refs, K=K, F=F, D=D, B=B),
        grid=(G,),
        in_specs=[
            pl.BlockSpec((P, 1), lambda k: (0, 0)),
            pl.BlockSpec((Pw // 128, 128), lambda k: (0, 0)),
            pl.BlockSpec((K, F), lambda k: (k, 0)),
            pl.BlockSpec((D, K), lambda k: (0, k)),
            pl.BlockSpec((F + D, D), lambda k: (0, 0)),
            pl.BlockSpec((1, D), lambda k: (0, 0)),
        ],
        out_specs=pl.BlockSpec((B_pad, D), lambda k: (0, 0)),
        out_shape=jax.ShapeDtypeStruct((B_pad, D), jnp.float32),
        compiler_params=pltpu.CompilerParams(
            dimension_semantics=("arbitrary",),
        ),
    )(ptr_pad, ptr_wide, x, h_node.T, W, b.reshape(1, D))
    return out[:B]


# speedup vs baseline: 1.0073x; 1.0073x over previous
import jax
import jax.numpy as jnp
from jax.experimental import pallas as pl
from jax.experimental.pallas import tpu as pltpu

_SMAX = 32
_NSTATIC = 3


def _pick_block(n):
    for cand in (5120, 2560, 1280, 640, 320, 160, 80, 40, 16, 8):
        if n % cand == 0:
            return cand
    return n


def _body(ptr_ref, ptrw_ref, x_ref, h_ref, w_ref, b_ref, out_ref, *, K, F, D, B):
    k = pl.program_id(0)
    r0 = k * K
    z = jnp.dot(
        x_ref[...].astype(jnp.bfloat16),
        w_ref[:F].astype(jnp.bfloat16),
        preferred_element_type=jnp.float32,
    )
    z = z + jax.lax.dot_general(
        h_ref[...], w_ref[F:],
        dimension_numbers=(((0,), (0,)), ((), ())),
        preferred_element_type=jnp.float32,
    )
    z = z + b_ref[...]
    z = jnp.where(z >= 0, z, 0.2 * z)

    ptr_wide = ptrw_ref[...]

    def seg_of(r):
        return jnp.sum((ptr_wide <= r).astype(jnp.int32)) - 1

    s0 = seg_of(r0)
    s1 = seg_of(r0 + K - 1)
    nchunks = (s1 - s0 + _SMAX) // _SMAX

    zstart = seg_of(r0 - 1) + 1
    zend = s0 + jnp.maximum(nchunks, _NSTATIC) * _SMAX
    nz = (zend - zstart + _SMAX - 1) // _SMAX
    zzero = jnp.zeros((_SMAX, D), jnp.float32)

    def zchunk(m, carry):
        out_ref[pl.ds(zstart + m * _SMAX, _SMAX), :] = zzero
        return carry

    for m in range(_NSTATIC + 1):
        zchunk(m, 0)
    jax.lax.fori_loop(_NSTATIC + 1, nz, zchunk, 0)

    rows = r0 + jax.lax.broadcasted_iota(jnp.int32, (_SMAX + 1, K), 1)

    def chunk(j, carry):
        base = s0 + j * _SMAX
        bounds = ptr_ref[pl.ds(base, _SMAX + 1), :]
        step = (rows >= bounds).astype(jnp.float32)
        csum = jnp.dot(step, z, preferred_element_type=jnp.float32)
        part = csum[:_SMAX] - csum[1:]
        out_ref[pl.ds(base, _SMAX), :] += part
        return carry

    for j in range(_NSTATIC):
        chunk(j, 0)
    jax.lax.fori_loop(_NSTATIC, nchunks, chunk, 0)


def kernel(h_node, x, ptr, W, b):
    N, F = x.shape
    D = h_node.shape[1]
    B = ptr.shape[0] - 1
    K = _pick_block(N)
    G = N // K

    pad_rows = (_NSTATIC + 1) * _SMAX
    B_pad = -(B + pad_rows) % 8 + (B + pad_rows)
    P = -(B + 1 + pad_rows) % 8 + (B + 1 + pad_rows)
    ptr_pad = jnp.concatenate(
        [ptr.astype(jnp.int32), jnp.full((P - (B + 1),), N, jnp.int32)]
    ).reshape(P, 1)
    Pw = -(B + 1) % 1024 + (B + 1)
    ptr_wide = jnp.concatenate(
        [ptr.astype(jnp.int32), jnp.full((Pw - (B + 1),), N, jnp.int32)]
    ).reshape(Pw // 128, 128)

    out = pl.pallas_call(
        lambda *refs: _body(*refs, K=K, F=F, D=D, B=B),
        grid=(G,),
        in_specs=[
            pl.BlockSpec((P, 1), lambda k: (0, 0)),
            pl.BlockSpec((Pw // 128, 128), lambda k: (0, 0)),
            pl.BlockSpec((K, F), lambda k: (k, 0)),
            pl.BlockSpec((D, K), lambda k: (0, k)),
            pl.BlockSpec((F + D, D), lambda k: (0, 0)),
            pl.BlockSpec((1, D), lambda k: (0, 0)),
        ],
        out_specs=pl.BlockSpec((B_pad, D), lambda k: (0, 0)),
        out_shape=jax.ShapeDtypeStruct((B_pad, D), jnp.float32),
        compiler_params=pltpu.CompilerParams(
            dimension_semantics=("arbitrary",),
        ),
    )(ptr_pad, ptr_wide, x, h_node.T, W, b.reshape(1, D))
    return out[:B]
